# initial kernel scaffold (unmeasured)
import jax
import jax.numpy as jnp
from jax import lax
from jax.experimental import pallas as pl
from jax.experimental.pallas import tpu as pltpu

N_DEV = 4


def kernel(x, w_mat, scale_x, scale_w):
    m_per, k = x.shape
    _, n_total = w_mat.shape
    n_per = n_total // N_DEV

    my_pos = lax.axis_index("i")
    w_local = lax.dynamic_slice_in_dim(w_mat, my_pos * n_per, n_per, axis=1)
    s = (scale_x[0] * scale_w[0]).reshape(1, 1)

    def body(x_ref, w_ref, s_ref, out_ref, comm_ref, send_sems, recv_sems):
        my = lax.axis_index("i")
        left = (my - 1) % N_DEV
        right = (my + 1) % N_DEV

        barrier_sem = pltpu.get_barrier_semaphore()
        for nbr in (left, right):
            pl.semaphore_signal(
                barrier_sem, inc=1,
                device_id=(nbr,), device_id_type=pl.DeviceIdType.MESH,
            )
        pl.semaphore_wait(barrier_sem, 2)

        scale = s_ref[0, 0]

        def gemm_block(src, origin):
            acc = jnp.dot(src[...], w_ref[...],
                          preferred_element_type=jnp.int32)
            out_ref[pl.ds(origin * m_per, m_per), :] = (
                acc.astype(jnp.float32) * scale)

        rdma = pltpu.make_async_remote_copy(
            src_ref=x_ref,
            dst_ref=comm_ref.at[0],
            send_sem=send_sems.at[0],
            recv_sem=recv_sems.at[0],
            device_id=(right,),
            device_id_type=pl.DeviceIdType.MESH,
        )
        rdma.start()
        gemm_block(x_ref, my)

        for h in range(1, N_DEV - 1):
            rdma.wait()
            rdma = pltpu.make_async_remote_copy(
                src_ref=comm_ref.at[h - 1],
                dst_ref=comm_ref.at[h],
                send_sem=send_sems.at[h],
                recv_sem=recv_sems.at[h],
                device_id=(right,),
                device_id_type=pl.DeviceIdType.MESH,
            )
            rdma.start()
            gemm_block(comm_ref.at[h - 1], (my - h) % N_DEV)

        rdma.wait()
        gemm_block(comm_ref.at[N_DEV - 2], (my + 1) % N_DEV)

    out_shape = jax.ShapeDtypeStruct((N_DEV * m_per, n_per), jnp.float32)
    return pl.pallas_call(
        body,
        out_shape=out_shape,
        in_specs=[
            pl.BlockSpec(memory_space=pltpu.VMEM),
            pl.BlockSpec(memory_space=pltpu.VMEM),
            pl.BlockSpec(memory_space=pltpu.SMEM),
        ],
        out_specs=pl.BlockSpec(memory_space=pltpu.VMEM),
        scratch_shapes=[
            pltpu.VMEM((N_DEV - 1, m_per, k), x.dtype),
            pltpu.SemaphoreType.DMA((N_DEV - 1,)),
            pltpu.SemaphoreType.DMA((N_DEV - 1,)),
        ],
        compiler_params=pltpu.CompilerParams(collective_id=0),
    )(x, w_local, s)


# baseline (device time: 205319 ns/iter reference)
import jax
import jax.numpy as jnp
from jax import lax
from jax.experimental import pallas as pl
from jax.experimental.pallas import tpu as pltpu

N_DEV = 4


def kernel(x, w_mat, scale_x, scale_w):
    m_per, k = x.shape
    _, n_total = w_mat.shape
    n_per = n_total // N_DEV

    my_pos = lax.axis_index("i")
    w_local = lax.dynamic_slice_in_dim(w_mat, my_pos * n_per, n_per, axis=1)
    s = (scale_x[0] * scale_w[0]).reshape(1, 1)

    def body(x_ref, w_ref, s_ref, out_hbm,
             comm_ref, yblk, send_sems, recv_sems, copy_sems):
        my = lax.axis_index("i")
        left = (my - 1) % N_DEV
        right = (my + 1) % N_DEV

        barrier_sem = pltpu.get_barrier_semaphore()
        for nbr in (left, right):
            pl.semaphore_signal(
                barrier_sem, inc=1,
                device_id=(nbr,), device_id_type=pl.DeviceIdType.MESH,
            )
        pl.semaphore_wait(barrier_sem, 2)

        scale = s_ref[0, 0]
        copies = [None, None]

        def gemm_block(b, src, origin):
            slot = b % 2
            if copies[slot] is not None:
                copies[slot].wait()
            acc = jnp.dot(src[...], w_ref[...],
                          preferred_element_type=jnp.int32)
            yblk[slot, :, :] = acc.astype(jnp.float32) * scale
            cp = pltpu.make_async_copy(
                yblk.at[slot],
                out_hbm.at[pl.ds(origin * m_per, m_per), :],
                copy_sems.at[slot],
            )
            cp.start()
            copies[slot] = cp

        rdma = pltpu.make_async_remote_copy(
            src_ref=x_ref,
            dst_ref=comm_ref.at[0],
            send_sem=send_sems.at[0],
            recv_sem=recv_sems.at[0],
            device_id=(right,),
            device_id_type=pl.DeviceIdType.MESH,
        )
        rdma.start()
        gemm_block(0, x_ref, my)

        for h in range(1, N_DEV - 1):
            rdma.wait()
            rdma = pltpu.make_async_remote_copy(
                src_ref=comm_ref.at[h - 1],
                dst_ref=comm_ref.at[h],
                send_sem=send_sems.at[h],
                recv_sem=recv_sems.at[h],
                device_id=(right,),
                device_id_type=pl.DeviceIdType.MESH,
            )
            rdma.start()
            gemm_block(h, comm_ref.at[h - 1], (my - h) % N_DEV)

        rdma.wait()
        gemm_block(N_DEV - 1, comm_ref.at[N_DEV - 2], (my + 1) % N_DEV)

        copies[0].wait()
        copies[1].wait()

    out_shape = jax.ShapeDtypeStruct((N_DEV * m_per, n_per), jnp.float32)
    return pl.pallas_call(
        body,
        out_shape=out_shape,
        in_specs=[
            pl.BlockSpec(memory_space=pltpu.VMEM),
            pl.BlockSpec(memory_space=pltpu.VMEM),
            pl.BlockSpec(memory_space=pltpu.SMEM),
        ],
        out_specs=pl.BlockSpec(memory_space=pl.ANY),
        scratch_shapes=[
            pltpu.VMEM((N_DEV - 1, m_per, k), x.dtype),
            pltpu.VMEM((2, m_per, n_per), jnp.float32),
            pltpu.SemaphoreType.DMA((N_DEV - 1,)),
            pltpu.SemaphoreType.DMA((N_DEV - 1,)),
            pltpu.SemaphoreType.DMA((2,)),
        ],
        compiler_params=pltpu.CompilerParams(
            collective_id=0,
            vmem_limit_bytes=60 * 1024 * 1024,
        ),
    )(x, w_local, s)


# device time: 152699 ns/iter; 1.3446x vs baseline; 1.3446x over previous
import jax
import jax.numpy as jnp
from jax import lax
from jax.experimental import pallas as pl
from jax.experimental.pallas import tpu as pltpu

N_DEV = 4

CW1, CCW1, CW2, CCW2 = 0, 1, 2, 3


def kernel(x, w_mat, scale_x, scale_w):
    m_per, k = x.shape
    _, n_total = w_mat.shape
    n_per = n_total // N_DEV
    half = m_per // 2

    my_pos = lax.axis_index("i")
    w_local = lax.dynamic_slice_in_dim(w_mat, my_pos * n_per, n_per, axis=1)
    s = (scale_x[0] * scale_w[0]).reshape(1, 1)

    def body(x_ref, w_ref, s_ref, out_hbm,
             cw0, ccw0, diag, yblk, send_sems, recv_sems, copy_sems):
        my = lax.axis_index("i")
        left = (my - 1) % N_DEV
        right = (my + 1) % N_DEV

        barrier_sem = pltpu.get_barrier_semaphore()
        for nbr in (left, right):
            pl.semaphore_signal(
                barrier_sem, inc=1,
                device_id=(nbr,), device_id_type=pl.DeviceIdType.MESH,
            )
        pl.semaphore_wait(barrier_sem, 2)

        scale = s_ref[0, 0]
        copies = [None, None]

        def gemm_block(b, src, origin):
            slot = b % 2
            if copies[slot] is not None:
                copies[slot].wait()
            acc = jnp.dot(src[...], w_ref[...],
                          preferred_element_type=jnp.int32)
            yblk[slot, :, :] = acc.astype(jnp.float32) * scale
            cp = pltpu.make_async_copy(
                yblk.at[slot],
                out_hbm.at[pl.ds(origin * m_per, m_per), :],
                copy_sems.at[slot],
            )
            cp.start()
            copies[slot] = cp

        def rdma(src, dst, sem, target):
            return pltpu.make_async_remote_copy(
                src_ref=src, dst_ref=dst,
                send_sem=send_sems.at[sem], recv_sem=recv_sems.at[sem],
                device_id=(target,), device_id_type=pl.DeviceIdType.MESH,
            )

        cw1 = rdma(x_ref, cw0, CW1, right)
        ccw1 = rdma(x_ref, ccw0, CCW1, left)
        cw1.start()
        ccw1.start()

        gemm_block(0, x_ref, my)

        cw1.wait()
        cw2 = rdma(cw0.at[pl.ds(0, half)], diag.at[pl.ds(0, half)],
                   CW2, right)
        cw2.start()
        ccw1.wait()
        ccw2 = rdma(ccw0.at[pl.ds(half, half)], diag.at[pl.ds(half, half)],
                    CCW2, left)
        ccw2.start()

        gemm_block(1, cw0, left)
        gemm_block(2, ccw0, right)

        cw2.wait()
        ccw2.wait()
        gemm_block(3, diag, (my + 2) % N_DEV)

        copies[0].wait()
        copies[1].wait()

    out_shape = jax.ShapeDtypeStruct((N_DEV * m_per, n_per), jnp.float32)
    return pl.pallas_call(
        body,
        out_shape=out_shape,
        in_specs=[
            pl.BlockSpec(memory_space=pltpu.VMEM),
            pl.BlockSpec(memory_space=pltpu.VMEM),
            pl.BlockSpec(memory_space=pltpu.SMEM),
        ],
        out_specs=pl.BlockSpec(memory_space=pl.ANY),
        scratch_shapes=[
            pltpu.VMEM((m_per, k), x.dtype),
            pltpu.VMEM((m_per, k), x.dtype),
            pltpu.VMEM((m_per, k), x.dtype),
            pltpu.VMEM((2, m_per, n_per), jnp.float32),
            pltpu.SemaphoreType.DMA((4,)),
            pltpu.SemaphoreType.DMA((4,)),
            pltpu.SemaphoreType.DMA((2,)),
        ],
        compiler_params=pltpu.CompilerParams(
            collective_id=0,
            vmem_limit_bytes=60 * 1024 * 1024,
        ),
    )(x, w_local, s)


# device time: 121040 ns/iter; 1.6963x vs baseline; 1.2616x over previous
import jax
import jax.numpy as jnp
from jax import lax
from jax.experimental import pallas as pl
from jax.experimental.pallas import tpu as pltpu

N_DEV = 4

CWA, CWB, CCWA, CCWB, CW2, CCW2 = range(6)


def kernel(x, w_mat, scale_x, scale_w):
    m_per, k = x.shape
    _, n_total = w_mat.shape
    n_per = n_total // N_DEV
    half = m_per // 2

    my_pos = lax.axis_index("i")
    s = (scale_x[0] * scale_w[0]).reshape(1, 1)

    def body(x_ref, w_hbm, s_ref, out_hbm,
             w_ref, cw0, ccw0, diag, yblk,
             send_sems, recv_sems, copy_sems, w_sem):
        my = lax.axis_index("i")
        left = (my - 1) % N_DEV
        right = (my + 1) % N_DEV

        w_load = pltpu.make_async_copy(
            w_hbm.at[:, pl.ds(my * n_per, n_per)], w_ref, w_sem)
        w_load.start()

        barrier_sem = pltpu.get_barrier_semaphore()
        for nbr in (left, right):
            pl.semaphore_signal(
                barrier_sem, inc=1,
                device_id=(nbr,), device_id_type=pl.DeviceIdType.MESH,
            )
        pl.semaphore_wait(barrier_sem, 2)

        scale = s_ref[0, 0]
        copies = [None, None]

        def gemm_rows(b, src, out_row, rows):
            slot = b % 2
            if copies[slot] is not None:
                copies[slot].wait()
            acc = jnp.dot(src[...], w_ref[...],
                          preferred_element_type=jnp.int32)
            yblk[slot, pl.ds(0, rows), :] = acc.astype(jnp.float32) * scale
            cp = pltpu.make_async_copy(
                yblk.at[slot, pl.ds(0, rows)],
                out_hbm.at[pl.ds(out_row, rows), :],
                copy_sems.at[slot],
            )
            cp.start()
            copies[slot] = cp

        def rdma(src, dst, sem, target):
            return pltpu.make_async_remote_copy(
                src_ref=src, dst_ref=dst,
                send_sem=send_sems.at[sem], recv_sem=recv_sems.at[sem],
                device_id=(target,), device_id_type=pl.DeviceIdType.MESH,
            )

        lo = pl.ds(0, half)
        hi = pl.ds(half, half)

        cwa = rdma(x_ref.at[lo], cw0.at[lo], CWA, right)
        cwb = rdma(x_ref.at[hi], cw0.at[hi], CWB, right)
        ccwa = rdma(x_ref.at[hi], ccw0.at[hi], CCWA, left)
        ccwb = rdma(x_ref.at[lo], ccw0.at[lo], CCWB, left)
        cwa.start()
        cwb.start()
        ccwa.start()
        ccwb.start()

        w_load.wait()
        gemm_rows(0, x_ref, my * m_per, m_per)

        cwa.wait()
        cw2 = rdma(cw0.at[lo], diag.at[lo], CW2, right)
        cw2.start()
        ccwa.wait()
        ccw2 = rdma(ccw0.at[hi], diag.at[hi], CCW2, left)
        ccw2.start()

        gemm_rows(1, cw0.at[lo], left * m_per, half)
        gemm_rows(2, ccw0.at[hi], right * m_per + half, half)

        cwb.wait()
        gemm_rows(3, cw0.at[hi], left * m_per + half, half)
        ccwb.wait()
        gemm_rows(0, ccw0.at[lo], right * m_per, half)

        cw2.wait()
        ccw2.wait()
        gemm_rows(1, diag, ((my + 2) % N_DEV) * m_per, m_per)

        copies[0].wait()
        copies[1].wait()

    out_shape = jax.ShapeDtypeStruct((N_DEV * m_per, n_per), jnp.float32)
    return pl.pallas_call(
        body,
        out_shape=out_shape,
        in_specs=[
            pl.BlockSpec(memory_space=pltpu.VMEM),
            pl.BlockSpec(memory_space=pl.ANY),
            pl.BlockSpec(memory_space=pltpu.SMEM),
        ],
        out_specs=pl.BlockSpec(memory_space=pl.ANY),
        scratch_shapes=[
            pltpu.VMEM((k, n_per), x.dtype),
            pltpu.VMEM((m_per, k), x.dtype),
            pltpu.VMEM((m_per, k), x.dtype),
            pltpu.VMEM((m_per, k), x.dtype),
            pltpu.VMEM((2, m_per, n_per), jnp.float32),
            pltpu.SemaphoreType.DMA((6,)),
            pltpu.SemaphoreType.DMA((6,)),
            pltpu.SemaphoreType.DMA((2,)),
            pltpu.SemaphoreType.DMA,
        ],
        compiler_params=pltpu.CompilerParams(
            collective_id=0,
            vmem_limit_bytes=60 * 1024 * 1024,
        ),
    )(x, w_mat, s)


# device time: 113436 ns/iter; 1.8100x vs baseline; 1.0670x over previous
import jax
import jax.numpy as jnp
from jax import lax
from jax.experimental import pallas as pl
from jax.experimental.pallas import tpu as pltpu

N_DEV = 4

CWA, CWB, CCWA, CCWB, CW2A, CW2B, CCW2A, CCW2B = range(8)


def kernel(x, w_mat, scale_x, scale_w):
    m_per, k = x.shape
    _, n_total = w_mat.shape
    n_per = n_total // N_DEV
    half = m_per // 2

    my_pos = lax.axis_index("i")
    s = (scale_x[0] * scale_w[0]).reshape(1, 1)

    def body(x_ref, w_hbm, s_ref, out_hbm,
             w_ref, cw0, ccw0, diag, yblk,
             send_sems, recv_sems, copy_sems, w_sem):
        my = lax.axis_index("i")
        left = (my - 1) % N_DEV
        right = (my + 1) % N_DEV

        w_load = pltpu.make_async_copy(
            w_hbm.at[:, pl.ds(my * n_per, n_per)], w_ref, w_sem)
        w_load.start()

        barrier_sem = pltpu.get_barrier_semaphore()
        for nbr in (left, right):
            pl.semaphore_signal(
                barrier_sem, inc=1,
                device_id=(nbr,), device_id_type=pl.DeviceIdType.MESH,
            )
        pl.semaphore_wait(barrier_sem, 2)

        scale = s_ref[0, 0]
        copies = [None, None]

        def gemm_rows(b, src, out_row, rows):
            slot = b % 2
            if copies[slot] is not None:
                copies[slot].wait()
            acc = jnp.dot(src[...], w_ref[...],
                          preferred_element_type=jnp.int32)
            yblk[slot, pl.ds(0, rows), :] = acc.astype(jnp.float32) * scale
            cp = pltpu.make_async_copy(
                yblk.at[slot, pl.ds(0, rows)],
                out_hbm.at[pl.ds(out_row, rows), :],
                copy_sems.at[slot],
            )
            cp.start()
            copies[slot] = cp

        def rdma(src, dst, sem, target):
            return pltpu.make_async_remote_copy(
                src_ref=src, dst_ref=dst,
                send_sem=send_sems.at[sem], recv_sem=recv_sems.at[sem],
                device_id=(target,), device_id_type=pl.DeviceIdType.MESH,
            )

        lo = pl.ds(0, half)
        hi = pl.ds(half, half)

        cwa = rdma(x_ref.at[lo], cw0.at[lo], CWA, right)
        cwb = rdma(x_ref.at[hi], cw0.at[hi], CWB, right)
        ccwa = rdma(x_ref.at[hi], ccw0.at[hi], CCWA, left)
        ccwb = rdma(x_ref.at[lo], ccw0.at[lo], CCWB, left)
        cwa.start()
        cwb.start()
        ccwa.start()
        ccwb.start()

        w_load.wait()
        gemm_rows(0, x_ref, my * m_per, m_per)

        quarter = m_per // 4
        q0, q1 = pl.ds(0, quarter), pl.ds(quarter, quarter)
        q2, q3 = pl.ds(half, quarter), pl.ds(half + quarter, quarter)

        cwa.wait()
        cw2a = rdma(cw0.at[q0], diag.at[q0], CW2A, right)
        cw2b = rdma(cw0.at[q1], diag.at[q1], CW2B, right)
        cw2a.start()
        cw2b.start()
        ccwa.wait()
        ccw2a = rdma(ccw0.at[q2], diag.at[q2], CCW2A, left)
        ccw2b = rdma(ccw0.at[q3], diag.at[q3], CCW2B, left)
        ccw2a.start()
        ccw2b.start()

        gemm_rows(1, cw0.at[lo], left * m_per, half)
        gemm_rows(2, ccw0.at[hi], right * m_per + half, half)

        cwb.wait()
        gemm_rows(3, cw0.at[hi], left * m_per + half, half)
        ccwb.wait()
        gemm_rows(0, ccw0.at[lo], right * m_per, half)

        d2 = ((my + 2) % N_DEV) * m_per
        cw2a.wait()
        gemm_rows(1, diag.at[q0], d2, quarter)
        ccw2a.wait()
        gemm_rows(0, diag.at[q2], d2 + half, quarter)
        cw2b.wait()
        gemm_rows(1, diag.at[q1], d2 + quarter, quarter)
        ccw2b.wait()
        gemm_rows(0, diag.at[q3], d2 + half + quarter, quarter)

        copies[0].wait()
        copies[1].wait()

    out_shape = jax.ShapeDtypeStruct((N_DEV * m_per, n_per), jnp.float32)
    return pl.pallas_call(
        body,
        out_shape=out_shape,
        in_specs=[
            pl.BlockSpec(memory_space=pltpu.VMEM),
            pl.BlockSpec(memory_space=pl.ANY),
            pl.BlockSpec(memory_space=pltpu.SMEM),
        ],
        out_specs=pl.BlockSpec(memory_space=pl.ANY),
        scratch_shapes=[
            pltpu.VMEM((k, n_per), x.dtype),
            pltpu.VMEM((m_per, k), x.dtype),
            pltpu.VMEM((m_per, k), x.dtype),
            pltpu.VMEM((m_per, k), x.dtype),
            pltpu.VMEM((2, m_per, n_per), jnp.float32),
            pltpu.SemaphoreType.DMA((8,)),
            pltpu.SemaphoreType.DMA((8,)),
            pltpu.SemaphoreType.DMA((2,)),
            pltpu.SemaphoreType.DMA,
        ],
        compiler_params=pltpu.CompilerParams(
            collective_id=0,
            vmem_limit_bytes=60 * 1024 * 1024,
        ),
    )(x, w_mat, s)


# device time: 107895 ns/iter; 1.9030x vs baseline; 1.0514x over previous
import jax
import jax.numpy as jnp
from jax import lax
from jax.experimental import pallas as pl
from jax.experimental.pallas import tpu as pltpu

N_DEV = 4

(CW0, CW1, CW2, CW3,
 XCW0, XCW1, XCW2, XCW3,
 FCWA, FCWB,
 FCCWA, FCCWB) = range(12)


def kernel(x, w_mat, scale_x, scale_w):
    m_per, k = x.shape
    _, n_total = w_mat.shape
    n_per = n_total // N_DEV
    half = m_per // 2
    quarter = m_per // 4

    s = (scale_x[0] * scale_w[0]).reshape(1, 1)

    def body(x_ref, w_hbm, s_ref, out_hbm,
             w_ref, cw0, ccw0, diag, yblk,
             send_sems, recv_sems, copy_sems, w_sem):
        my = lax.axis_index("i")
        left = (my - 1) % N_DEV
        right = (my + 1) % N_DEV

        w_load = pltpu.make_async_copy(
            w_hbm.at[:, pl.ds(my * n_per, n_per)], w_ref, w_sem)
        w_load.start()

        barrier_sem = pltpu.get_barrier_semaphore()
        for nbr in (left, right):
            pl.semaphore_signal(
                barrier_sem, inc=1,
                device_id=(nbr,), device_id_type=pl.DeviceIdType.MESH,
            )
        pl.semaphore_wait(barrier_sem, 2)

        scale = s_ref[0, 0]
        copies = [None, None]
        bctr = [0]

        def gemm_rows(src, out_row, rows):
            slot = bctr[0] % 2
            bctr[0] += 1
            if copies[slot] is not None:
                copies[slot].wait()
            acc = jnp.dot(src[...], w_ref[...],
                          preferred_element_type=jnp.int32)
            yblk[slot, pl.ds(0, rows), :] = acc.astype(jnp.float32) * scale
            cp = pltpu.make_async_copy(
                yblk.at[slot, pl.ds(0, rows)],
                out_hbm.at[pl.ds(out_row, rows), :],
                copy_sems.at[slot],
            )
            cp.start()
            copies[slot] = cp

        def rdma(src, dst, sem, target):
            return pltpu.make_async_remote_copy(
                src_ref=src, dst_ref=dst,
                send_sem=send_sems.at[sem], recv_sem=recv_sems.at[sem],
                device_id=(target,), device_id_type=pl.DeviceIdType.MESH,
            )

        q = [pl.ds(i * quarter, quarter) for i in range(4)]

        cw = [rdma(x_ref.at[q[i]], cw0.at[q[i]], CW0 + i, right)
              for i in range(4)]
        ccw = [rdma(x_ref.at[q[i]], ccw0.at[q[i]], XCW0 + i, left)
               for i in range(4)]
        for r in cw:
            r.start()
        for r in (ccw[3], ccw[2], ccw[1], ccw[0]):
            r.start()

        w_load.wait()
        gemm_rows(x_ref, my * m_per, m_per)

        cw[0].wait()
        fcwa = rdma(cw0.at[q[0]], diag.at[q[0]], FCWA, right)
        fcwa.start()
        ccw[3].wait()
        fccwa = rdma(ccw0.at[q[3]], diag.at[q[3]], FCCWA, left)
        fccwa.start()
        gemm_rows(cw0.at[q[0]], left * m_per, quarter)
        gemm_rows(ccw0.at[q[3]], right * m_per + 3 * quarter, quarter)

        cw[1].wait()
        fcwb = rdma(cw0.at[q[1]], diag.at[q[1]], FCWB, right)
        fcwb.start()
        ccw[2].wait()
        fccwb = rdma(ccw0.at[q[2]], diag.at[q[2]], FCCWB, left)
        fccwb.start()
        gemm_rows(cw0.at[q[1]], left * m_per + quarter, quarter)
        gemm_rows(ccw0.at[q[2]], right * m_per + half, quarter)

        cw[2].wait()
        gemm_rows(cw0.at[q[2]], left * m_per + half, quarter)
        ccw[1].wait()
        gemm_rows(ccw0.at[q[1]], right * m_per + quarter, quarter)
        cw[3].wait()
        gemm_rows(cw0.at[q[3]], left * m_per + 3 * quarter, quarter)
        ccw[0].wait()
        gemm_rows(ccw0.at[q[0]], right * m_per, quarter)

        d2 = ((my + 2) % N_DEV) * m_per
        fcwa.wait()
        gemm_rows(diag.at[q[0]], d2, quarter)
        fccwa.wait()
        gemm_rows(diag.at[q[3]], d2 + 3 * quarter, quarter)
        fcwb.wait()
        gemm_rows(diag.at[q[1]], d2 + quarter, quarter)
        fccwb.wait()
        gemm_rows(diag.at[q[2]], d2 + half, quarter)

        copies[0].wait()
        copies[1].wait()

    out_shape = jax.ShapeDtypeStruct((N_DEV * m_per, n_per), jnp.float32)
    return pl.pallas_call(
        body,
        out_shape=out_shape,
        in_specs=[
            pl.BlockSpec(memory_space=pltpu.VMEM),
            pl.BlockSpec(memory_space=pl.ANY),
            pl.BlockSpec(memory_space=pltpu.SMEM),
        ],
        out_specs=pl.BlockSpec(memory_space=pl.ANY),
        scratch_shapes=[
            pltpu.VMEM((k, n_per), x.dtype),
            pltpu.VMEM((m_per, k), x.dtype),
            pltpu.VMEM((m_per, k), x.dtype),
            pltpu.VMEM((m_per, k), x.dtype),
            pltpu.VMEM((2, m_per, n_per), jnp.float32),
            pltpu.SemaphoreType.DMA((12,)),
            pltpu.SemaphoreType.DMA((12,)),
            pltpu.SemaphoreType.DMA((2,)),
            pltpu.SemaphoreType.DMA,
        ],
        compiler_params=pltpu.CompilerParams(
            collective_id=0,
            vmem_limit_bytes=60 * 1024 * 1024,
        ),
    )(x, w_mat, s)
